# baseline (speedup 1.0000x reference)
import jax
import jax.numpy as jnp
from jax import lax
from jax.experimental import pallas as pl
from jax.experimental.pallas import tpu as pltpu
from jax.experimental.pallas import tpu_sc as plsc

_NC = 2
_NS = 16
_NW = _NC * _NS

_B = 4
_T = 2048
_D = 128
_TOTAL = _B * _T
_PER_W = _TOTAL // _NW
_CHUNK = 128
_NCHUNK = _PER_W // _CHUNK


def _emb_kernel(ids_hbm, tok_hbm, pos_hbm, out_hbm, idx_v, buf_v,
                sem_i, sem_p, sem_g, sem_w):
    c = lax.axis_index("c")
    s = lax.axis_index("s")
    wid = s * _NC + c
    base = wid * _PER_W
    b = base // _T
    t_base = lax.rem(base, _T)

    idx_dma = pltpu.async_copy(
        ids_hbm.at[b, pl.ds(t_base, _PER_W)], idx_v, sem_i)
    pos_dma = pltpu.async_copy(
        pos_hbm.at[pl.ds(t_base, _PER_W)], buf_v, sem_p)

    idx_dma.wait()
    pos_dma.wait()
    gathers = [
        pltpu.async_copy(
            tok_hbm.at[idx_v.at[pl.ds(j * _CHUNK, _CHUNK)]],
            buf_v.at[pl.ds(j * _CHUNK, _CHUNK)],
            sem_g[j],
            add=True,
        )
        for j in range(_NCHUNK)
    ]

    writes = []
    for j in range(_NCHUNK):
        gathers[j].wait()
        writes.append(
            pltpu.async_copy(
                buf_v.at[pl.ds(j * _CHUNK, _CHUNK)],
                out_hbm.at[pl.ds(base + j * _CHUNK, _CHUNK)],
                sem_w[j],
            )
        )
    for w in writes:
        w.wait()


@jax.jit
def kernel(input_ids, token_table, pos_table):
    mesh = plsc.VectorSubcoreMesh(core_axis_name="c", subcore_axis_name="s")
    out = pl.kernel(
        _emb_kernel,
        out_type=jax.ShapeDtypeStruct((_TOTAL, _D), jnp.float32),
        mesh=mesh,
        scratch_types=[
            pltpu.VMEM((_PER_W,), jnp.int32),
            pltpu.VMEM((_PER_W, _D), jnp.float32),
            pltpu.SemaphoreType.DMA,
            pltpu.SemaphoreType.DMA,
            [pltpu.SemaphoreType.DMA] * _NCHUNK,
            [pltpu.SemaphoreType.DMA] * _NCHUNK,
        ],
    )(input_ids, token_table, pos_table)
    return out.reshape(_B, _T, _D)
